# Initial kernel scaffold; baseline (speedup 1.0000x reference)
import jax
import jax.numpy as jnp
from jax.experimental import pallas as pl


def kernel(x, edge_index, batch, W1, b1, g1, be1, W2, b2, g2, be2, W3, b3, g3, be3, Wg, att_src, att_dst, bg, Wfc, bfc):
    raise NotImplementedError("write your pallas kernel here")


# speedup vs baseline: 1.0021x; 1.0021x over previous
import functools

import jax
import jax.numpy as jnp
from jax.experimental import pallas as pl
from jax.experimental.pallas import tpu as pltpu

N = 10000
E = 320000
D_IN = 128
HID = 128
OUT = 2 * HID
GAT_OUT = 256
HEADS = 4
N_GRAPHS = 64
FUSED = 512


def _pool_fc_body(z_ref, batch_ref, wfc_ref, bfc_ref, out_ref):
    z = z_ref[:, :]
    b = batch_ref[:, :].astype(jnp.int32)
    gids = jax.lax.broadcasted_iota(jnp.int32, (N_GRAPHS, N), 0)
    mask = (b == gids).astype(jnp.float32)
    sums = jnp.dot(mask, z, preferred_element_type=jnp.float32)
    cnt = jnp.sum(mask, axis=1, keepdims=True)
    pooled = sums / jnp.maximum(cnt, 1.0)
    out = jnp.dot(pooled, wfc_ref[:, :], preferred_element_type=jnp.float32)
    out_ref[:, :] = jnp.maximum(out + bfc_ref[:, :], 0.0)


def _pool_fc(z, batch, Wfc, bfc):
    return pl.pallas_call(
        _pool_fc_body,
        out_shape=jax.ShapeDtypeStruct((N_GRAPHS, FUSED), jnp.float32),
    )(z, batch.reshape(1, N), Wfc, bfc.reshape(1, FUSED))


def _gcn_conv(x, src, dst, W, b, dinv):
    h = x @ W
    out = jnp.zeros((N, W.shape[1]), x.dtype).at[dst].add(h[src] * (dinv[src] * dinv[dst])[:, None])
    return out + b


def _bn(x, gamma, beta):
    m = jnp.mean(x, axis=0)
    v = jnp.var(x, axis=0)
    return (x - m) / jnp.sqrt(v + 1e-5) * gamma + beta


def _gat_conv(x, src, dst, Wg, att_src, att_dst, bg):
    h = (x @ Wg).reshape(N, HEADS, GAT_OUT)
    a_s = jnp.sum(h * att_src, axis=-1)
    a_d = jnp.sum(h * att_dst, axis=-1)
    e = a_s[src] + a_d[dst]
    e = jax.nn.leaky_relu(e, 0.2)
    emax = jnp.full((N, HEADS), -1e30, x.dtype).at[dst].max(e)
    ex = jnp.exp(e - emax[dst])
    den = jnp.zeros((N, HEADS), x.dtype).at[dst].add(ex)
    coef = ex / (den[dst] + 1e-16)
    out = jnp.zeros((N, HEADS, GAT_OUT), x.dtype).at[dst].add(h[src] * coef[:, :, None])
    return jnp.mean(out, axis=1) + bg


def kernel(x, edge_index, batch, W1, b1, g1, be1, W2, b2, g2, be2,
           W3, b3, g3, be3, Wg, att_src, att_dst, bg, Wfc, bfc):
    loop = jnp.arange(N, dtype=edge_index.dtype)
    src = jnp.concatenate([edge_index[0], loop])
    dst = jnp.concatenate([edge_index[1], loop])
    deg = jnp.zeros((N,), x.dtype).at[dst].add(1.0)
    dinv = jnp.where(deg > 0, 1.0 / jnp.sqrt(deg), 0.0)
    h = _gcn_conv(x, src, dst, W1, b1, dinv)
    h = jax.nn.relu(_bn(h, g1, be1))
    h = _gcn_conv(h, src, dst, W2, b2, dinv)
    h = jax.nn.relu(_bn(h, g2, be2))
    h = _gcn_conv(h, src, dst, W3, b3, dinv)
    h = jax.nn.relu(_bn(h, g3, be3))
    h = _gat_conv(h, src, dst, Wg, att_src, att_dst, bg)
    h = jax.nn.relu(_bn(h, g3, be3))
    return _pool_fc(h, batch, Wfc, bfc)


# speedup vs baseline: 1.5079x; 1.5047x over previous
import functools

import jax
import jax.numpy as jnp
from jax import lax
from jax.experimental import pallas as pl
from jax.experimental.pallas import tpu as pltpu
from jax.experimental.pallas import tpu_sc as plsc

N = 10000
E = 320000
D_IN = 128
HID = 128
OUT = 2 * HID
GAT_OUT = 256
HEADS = 4
N_GRAPHS = 64
FUSED = 512

N_PAD = 10112
TRASH = 10008
E_TOT = E + N
K = 128
E_PAD = 331776

NPASS = 8
NH = 1264
ACC_R = 1280
RPS = ACC_R // 16

_MESH = plsc.VectorSubcoreMesh(core_axis_name="c", subcore_axis_name="s")


def _fill_buf(buf, rows, dh, val):
    zv = jnp.full((16,), val, jnp.float32)
    for j in range(dh // 16):
        def body(i, _, j=j):
            buf[i, j * 16:(j + 1) * 16] = zv
            return 0
        lax.fori_loop(0, rows, body, 0)


def _zero_acc_rows(zbuf, acc, row0):
    pltpu.sync_copy(zbuf.at[pl.ds(0, RPS)], acc.at[pl.ds(row0, RPS)])


def _remap_dst(didx_raw, didx, half):
    off = half * NH
    for j in range(8):
        d = didx_raw[j * 16:(j + 1) * 16] - off
        ok = (d >= 0) & (d < NH)
        didx[j * 16:(j + 1) * 16] = jnp.where(ok, d, NH)


def _sc_deg_body(dst_hbm, out_hbm, draw, didx, ones_v, acc):
    c = lax.axis_index("c")
    s = lax.axis_index("s")
    row0 = s * RPS
    base0 = (c * 16 + s) * (E_PAD // 32)
    for half in range(NPASS):
        _fill_buf(ones_v, 128, 128, 0.0)
        _zero_acc_rows(ones_v, acc, row0)
        _fill_buf(ones_v, K, 128, 1.0)
        plsc.subcore_barrier()

        def chunk(ch, _):
            base = base0 + ch * K
            pltpu.sync_copy(dst_hbm.at[pl.ds(base, K)], draw)
            _remap_dst(draw, didx, half)
            pltpu.sync_copy(ones_v, acc.at[didx], add=True)
            return 0
        lax.fori_loop(0, E_PAD // 32 // K, chunk, 0)
        plsc.subcore_barrier()
        pltpu.sync_copy(acc.at[pl.ds(row0, RPS)],
                        out_hbm.at[pl.ds((c * NPASS + half) * ACC_R + row0, RPS)])


_sc_deg = functools.partial(
    pl.kernel,
    out_type=jax.ShapeDtypeStruct((2 * NPASS * ACC_R, 128), jnp.float32),
    mesh=_MESH,
    scratch_types=[
        pltpu.VMEM((K,), jnp.int32),
        pltpu.VMEM((K,), jnp.int32),
        pltpu.VMEM((K, 128), jnp.float32),
        pltpu.VMEM_SHARED((ACC_R, 128), jnp.float32),
    ],
)(_sc_deg_body)


def _make_sc_agg(feat_split):
    def body(src_hbm, dst_hbm, tab_hbm, out_hbm, sidx, draw, didx, vals, zb, acc):
        c = lax.axis_index("c")
        s = lax.axis_index("s")
        row0 = s * RPS
        _fill_buf(zb, 128, 128, 0.0)
        if feat_split:
            base0 = s * (E_PAD // 16)
            nch = E_PAD // 16 // K
            sbase0 = c * E_PAD + base0
        else:
            base0 = (c * 16 + s) * (E_PAD // 32)
            nch = E_PAD // 32 // K
            sbase0 = base0

        for half in range(NPASS):
            _zero_acc_rows(zb, acc, row0)
            plsc.subcore_barrier()

            def chunk(ch, _):
                base = base0 + ch * K
                sbase = sbase0 + ch * K
                pltpu.sync_copy(src_hbm.at[pl.ds(sbase, K)], sidx)
                pltpu.sync_copy(dst_hbm.at[pl.ds(base, K)], draw)
                _remap_dst(draw, didx, half)
                pltpu.sync_copy(tab_hbm.at[sidx], vals)
                pltpu.sync_copy(vals, acc.at[didx], add=True)
                return 0
            lax.fori_loop(0, nch, chunk, 0)
            plsc.subcore_barrier()
            pltpu.sync_copy(acc.at[pl.ds(row0, RPS)],
                            out_hbm.at[pl.ds((c * NPASS + half) * ACC_R + row0, RPS)])

    return functools.partial(
        pl.kernel,
        out_type=jax.ShapeDtypeStruct((2 * NPASS * ACC_R, 128), jnp.float32),
        mesh=_MESH,
        scratch_types=[
            pltpu.VMEM((K,), jnp.int32),
            pltpu.VMEM((K,), jnp.int32),
            pltpu.VMEM((K,), jnp.int32),
            pltpu.VMEM((K, 128), jnp.float32),
            pltpu.VMEM((128, 128), jnp.float32),
            pltpu.VMEM_SHARED((ACC_R, 128), jnp.float32),
        ],
    )(body)


_sc_agg_edge = _make_sc_agg(False)
_sc_agg_feat = _make_sc_agg(True)


def _sc_att1_body(src_hbm, dst_hbm, as_hbm, ad_hbm, p_hbm, den_hbm,
                  sidx, draw, didx, arow_s, arow_d, pbuf, pbuf16, acc):
    c = lax.axis_index("c")
    s = lax.axis_index("s")
    row0 = s * RPS
    base0 = (c * 16 + s) * (E_PAD // 32)
    for half in range(NPASS):
        _fill_buf(pbuf, K, 128, 0.0)
        _zero_acc_rows(pbuf, acc, row0)
        plsc.subcore_barrier()

        def chunk(ch, _):
            base = base0 + ch * K
            pltpu.sync_copy(src_hbm.at[pl.ds(base, K)], sidx)
            pltpu.sync_copy(dst_hbm.at[pl.ds(base, K)], draw)
            _remap_dst(draw, didx, half)
            pltpu.sync_copy(as_hbm.at[sidx], arow_s)
            pltpu.sync_copy(ad_hbm.at[draw], arow_d)

            def ebody(i, _):
                e = arow_s[i, 0:16] + arow_d[i, 0:16]
                e = jnp.where(e >= 0.0, e, 0.2 * e)
                p = jnp.exp(e)
                pbuf[i, 0:16] = p
                pbuf16[i, 0:16] = p
                return 0
            lax.fori_loop(0, K, ebody, 0)
            if half == 0:
                pltpu.sync_copy(pbuf16, p_hbm.at[pl.ds(base, K)])
            pltpu.sync_copy(pbuf, acc.at[didx], add=True)
            return 0
        lax.fori_loop(0, E_PAD // 32 // K, chunk, 0)
        plsc.subcore_barrier()
        pltpu.sync_copy(acc.at[pl.ds(row0, RPS)],
                        den_hbm.at[pl.ds((c * NPASS + half) * ACC_R + row0, RPS)])


_sc_att1 = functools.partial(
    pl.kernel,
    out_type=[
        jax.ShapeDtypeStruct((E_PAD, 16), jnp.float32),
        jax.ShapeDtypeStruct((2 * NPASS * ACC_R, 128), jnp.float32),
    ],
    mesh=_MESH,
    scratch_types=[
        pltpu.VMEM((K,), jnp.int32),
        pltpu.VMEM((K,), jnp.int32),
        pltpu.VMEM((K,), jnp.int32),
        pltpu.VMEM((K, 128), jnp.float32),
        pltpu.VMEM((K, 128), jnp.float32),
        pltpu.VMEM((K, 128), jnp.float32),
        pltpu.VMEM((K, 16), jnp.float32),
        pltpu.VMEM_SHARED((ACC_R, 128), jnp.float32),
    ],
)(_sc_att1_body)


def _sc_att2_body(src_hbm, dst_hbm, htab_hbm, p_hbm, den_hbm, out_hbm,
                  sidx, draw, didx, hbuf, pbuf, drow, vbuf, acc):
    c = lax.axis_index("c")
    s = lax.axis_index("s")
    row0 = s * RPS
    base0 = s * (E_PAD // 16)
    for half in range(NPASS):
        _fill_buf(vbuf, K, 128, 0.0)
        _zero_acc_rows(vbuf, acc, row0)
        plsc.subcore_barrier()

        def chunk(ch, _):
            base = base0 + ch * K
            sbase = c * E_PAD + base
            pltpu.sync_copy(src_hbm.at[pl.ds(sbase, K)], sidx)
            pltpu.sync_copy(dst_hbm.at[pl.ds(base, K)], draw)
            _remap_dst(draw, didx, half)
            pltpu.sync_copy(htab_hbm.at[sidx], hbuf)
            pltpu.sync_copy(p_hbm.at[pl.ds(base, K)], pbuf)
            pltpu.sync_copy(den_hbm.at[draw], drow)

            def blend(i, _):
                cf = (0.25 * pbuf[i, 0:16]) / (drow[i, 0:16] + 1e-16)
                c0 = cf[0]
                c1 = cf[1]
                c2 = cf[2]
                c3 = cf[3]
                for j in range(8):
                    v = (c0 * hbuf[i, 0 + j * 16:0 + j * 16 + 16]
                         + c1 * hbuf[i, 128 + j * 16:128 + j * 16 + 16]
                         + c2 * hbuf[i, 256 + j * 16:256 + j * 16 + 16]
                         + c3 * hbuf[i, 384 + j * 16:384 + j * 16 + 16])
                    vbuf[i, j * 16:j * 16 + 16] = v
                return 0
            lax.fori_loop(0, K, blend, 0)
            pltpu.sync_copy(vbuf, acc.at[didx], add=True)
            return 0
        lax.fori_loop(0, E_PAD // 16 // K, chunk, 0)
        plsc.subcore_barrier()
        pltpu.sync_copy(acc.at[pl.ds(row0, RPS)],
                        out_hbm.at[pl.ds((c * NPASS + half) * ACC_R + row0, RPS)])


_sc_att2 = functools.partial(
    pl.kernel,
    out_type=jax.ShapeDtypeStruct((2 * NPASS * ACC_R, 128), jnp.float32),
    mesh=_MESH,
    scratch_types=[
        pltpu.VMEM((K,), jnp.int32),
        pltpu.VMEM((K,), jnp.int32),
        pltpu.VMEM((K,), jnp.int32),
        pltpu.VMEM((K, 512), jnp.float32),
        pltpu.VMEM((K, 16), jnp.float32),
        pltpu.VMEM((K, 128), jnp.float32),
        pltpu.VMEM((K, 128), jnp.float32),
        pltpu.VMEM_SHARED((ACC_R, 128), jnp.float32),
    ],
)(_sc_att2_body)


def _bn_relu(h, g, be):
    m = jnp.mean(h, axis=0, keepdims=True)
    d = h - m
    v = jnp.mean(d * d, axis=0, keepdims=True)
    return jnp.maximum(d / jnp.sqrt(v + 1e-5) * g + be, 0.0)


def _tc_first_body(deg_ref, x_ref, w_ref, dinv_ref, out_ref):
    deg = deg_ref[:, :]
    dinv = 1.0 / jnp.sqrt(deg)
    dinv_ref[:, :] = dinv
    hp = dinv * jnp.dot(x_ref[:, :], w_ref[:, :], preferred_element_type=jnp.float32)
    out_ref[0:N, :] = hp
    out_ref[N:N_PAD, :] = jnp.zeros((N_PAD - N, HID), jnp.float32)


def _tc_first(deg, x, W1):
    return pl.pallas_call(
        _tc_first_body,
        out_shape=[
            jax.ShapeDtypeStruct((N, 1), jnp.float32),
            jax.ShapeDtypeStruct((N_PAD, HID), jnp.float32),
        ],
    )(deg, x, W1)


def _make_tc_mid(d_in, d_out):
    def body(agg_ref, dinv_ref, b_ref, g_ref, be_ref, w_ref, out_ref):
        dinv = dinv_ref[:, :]
        h = dinv * agg_ref[:, :] + b_ref[:, :]
        y = _bn_relu(h, g_ref[:, :], be_ref[:, :])
        hp = dinv * jnp.dot(y, w_ref[:, :], preferred_element_type=jnp.float32)
        half = d_out // 2
        out_ref[0, 0:N, :] = hp[:, :half]
        out_ref[1, 0:N, :] = hp[:, half:]
        out_ref[0, N:N_PAD, :] = jnp.zeros((N_PAD - N, half), jnp.float32)
        out_ref[1, N:N_PAD, :] = jnp.zeros((N_PAD - N, half), jnp.float32)

    def run(agg, dinv, b, g, be, Wn):
        return pl.pallas_call(
            body,
            out_shape=jax.ShapeDtypeStruct((2, N_PAD, d_out // 2), jnp.float32),
        )(agg, dinv, b.reshape(1, d_in), g.reshape(1, d_in), be.reshape(1, d_in), Wn)
    return run


_tc_mid2 = _make_tc_mid(HID, OUT)
_tc_mid3 = _make_tc_mid(OUT, OUT)


def _tc_gatin_body(agg_ref, dinv_ref, b_ref, g_ref, be_ref, aw_ref,
                   y_ref, aa_ref):
    dinv = dinv_ref[:, :]
    h = dinv * agg_ref[:, :] + b_ref[:, :]
    y = _bn_relu(h, g_ref[:, :], be_ref[:, :])
    y_ref[:, :] = y
    aa_ref[:, :] = jnp.dot(y, aw_ref[:, :], preferred_element_type=jnp.float32)


def _tc_gatin(agg, dinv, b3, g3, be3, AsAd):
    return pl.pallas_call(
        _tc_gatin_body,
        out_shape=[
            jax.ShapeDtypeStruct((N, OUT), jnp.float32),
            jax.ShapeDtypeStruct((N, 32), jnp.float32),
        ],
    )(agg, dinv, b3.reshape(1, OUT), g3.reshape(1, OUT), be3.reshape(1, OUT), AsAd)


def _tc_hproj_body(y_ref, wg_ref, out_ref):
    out_ref[0, 0:N, :] = jnp.dot(y_ref[:, :], wg_ref[0], preferred_element_type=jnp.float32)
    out_ref[0, N:N_PAD, :] = jnp.zeros((N_PAD - N, 512), jnp.float32)


def _tc_hproj(y3, WgR):
    return pl.pallas_call(
        _tc_hproj_body,
        grid=(2,),
        in_specs=[
            pl.BlockSpec((N, OUT), lambda i: (0, 0)),
            pl.BlockSpec((1, OUT, 512), lambda i: (i, 0, 0)),
        ],
        out_specs=pl.BlockSpec((1, N_PAD, 512), lambda i: (i, 0, 0)),
        out_shape=jax.ShapeDtypeStruct((2, N_PAD, 512), jnp.float32),
    )(y3, WgR)


def _tc_final_body(t_ref, bg_ref, g_ref, be_ref, batch_ref, wfc_ref, bfc_ref,
                   out_ref):
    h = t_ref[:, :] + bg_ref[:, :]
    z = _bn_relu(h, g_ref[:, :], be_ref[:, :])
    b = batch_ref[:, :]
    gids = jax.lax.broadcasted_iota(jnp.int32, (N_GRAPHS, N), 0)
    mask = (b == gids).astype(jnp.float32)
    sums = jnp.dot(mask, z, preferred_element_type=jnp.float32)
    cnt = jnp.sum(mask, axis=1, keepdims=True)
    pooled = sums / jnp.maximum(cnt, 1.0)
    out = jnp.dot(pooled, wfc_ref[:, :], preferred_element_type=jnp.float32)
    out_ref[:, :] = jnp.maximum(out + bfc_ref[:, :], 0.0)


def _tc_final(t, bg, g3, be3, batch, Wfc, bfc):
    return pl.pallas_call(
        _tc_final_body,
        out_shape=jax.ShapeDtypeStruct((N_GRAPHS, FUSED), jnp.float32),
    )(t, bg.reshape(1, GAT_OUT), g3.reshape(1, GAT_OUT), be3.reshape(1, GAT_OUT),
      batch.reshape(1, N), Wfc, bfc.reshape(1, FUSED))


def _core_rows(o, c):
    parts = []
    for q in range(NPASS):
        nrows = min(NH, N - q * NH)
        base = (c * NPASS + q) * ACC_R
        parts.append(o[base:base + nrows])
    return jnp.concatenate(parts, axis=0)


def _recon_edge_split(o):
    return _core_rows(o, 0) + _core_rows(o, 1)


def _recon_feat_split(o):
    return jnp.concatenate([_core_rows(o, 0), _core_rows(o, 1)], axis=1)


def kernel(x, edge_index, batch, W1, b1, g1, be1, W2, b2, g2, be2,
           W3, b3, g3, be3, Wg, att_src, att_dst, bg, Wfc, bfc):
    loop = jnp.arange(N, dtype=edge_index.dtype)
    src = jnp.concatenate([edge_index[0], loop])
    dst = jnp.concatenate([edge_index[1], loop])
    src_p = jnp.full((E_PAD,), N, jnp.int32).at[:E_TOT].set(src)
    dst_p = jnp.full((E_PAD,), TRASH, jnp.int32).at[:E_TOT].set(dst)
    src2 = jnp.concatenate([src_p, src_p + N_PAD])

    Wg3 = Wg.reshape(OUT, HEADS, GAT_OUT)
    As = jnp.einsum("khd,hd->kh", Wg3, att_src)
    Ad = jnp.einsum("khd,hd->kh", Wg3, att_dst)
    AsAd = jnp.zeros((OUT, 32), jnp.float32).at[:, 0:4].set(As).at[:, 16:20].set(Ad)
    WgR = jnp.stack([Wg3[:, :, :128].reshape(OUT, 512),
                     Wg3[:, :, 128:].reshape(OUT, 512)])

    deg = _recon_edge_split(_sc_deg(dst_p))[:, 0:1]

    dinv, h1p = _tc_first(deg, x, W1)
    agg1 = _recon_edge_split(_sc_agg_edge(src_p, dst_p, h1p))

    h2p = _tc_mid2(agg1, dinv, b1, g1, be1, W2)
    agg2 = _recon_feat_split(_sc_agg_feat(src2, dst_p, h2p.reshape(2 * N_PAD, 128)))

    h3p = _tc_mid3(agg2, dinv, b2, g2, be2, W3)
    agg3 = _recon_feat_split(_sc_agg_feat(src2, dst_p, h3p.reshape(2 * N_PAD, 128)))

    y3, aa = _tc_gatin(agg3, dinv, b3, g3, be3, AsAd)
    as_tab = jnp.zeros((N_PAD, 128), jnp.float32).at[:N, 0:16].set(aa[:, 0:16])
    ad_tab = jnp.zeros((N_PAD, 128), jnp.float32).at[:N, 0:16].set(aa[:, 16:32])

    p_e, den4 = _sc_att1(src_p, dst_p, as_tab, ad_tab)
    den = jnp.zeros((N_PAD, 128), jnp.float32).at[:N].set(_recon_edge_split(den4))

    htab = _tc_hproj(y3, WgR)
    t = _recon_feat_split(
        _sc_att2(src2, dst_p, htab.reshape(2 * N_PAD, 512), p_e, den))

    return _tc_final(t, bg, g3, be3, batch, Wfc, bfc)


# speedup vs baseline: 7.4041x; 4.9103x over previous
import functools

import jax
import jax.numpy as jnp
from jax import lax
from jax.experimental import pallas as pl
from jax.experimental.pallas import tpu as pltpu
from jax.experimental.pallas import tpu_sc as plsc

N = 10000
E = 320000
D_IN = 128
HID = 128
OUT = 2 * HID
GAT_OUT = 256
HEADS = 4
N_GRAPHS = 64
FUSED = 512

N_PAD = 10112
TRASH = 10008
E_TOT = E + N
K = 128
E_PAD = 331776

NPASS = 8
NH = 1264
ACC_R = 1280
RPS = ACC_R // 16
MAXCH32 = (E_PAD // K + 31) // 32 + 1
MAXCH16 = (E_PAD // K + 15) // 16 + 1

_MESH = plsc.VectorSubcoreMesh(core_axis_name="c", subcore_axis_name="s")


def _fill_buf(buf, rows, dh, val):
    zv = jnp.full((16,), val, jnp.float32)
    for j in range(dh // 16):
        def body(i, _, j=j):
            buf[i, j * 16:(j + 1) * 16] = zv
            return 0
        lax.fori_loop(0, rows, body, 0)


def _zero_acc_rows(zbuf, acc, row0):
    pltpu.sync_copy(zbuf.at[pl.ds(0, RPS)], acc.at[pl.ds(row0, RPS)])


def _remap_dst(didx_raw, didx, half):
    off = half * NH
    for j in range(8):
        d = didx_raw[j * 16:(j + 1) * 16] - off
        ok = (d >= 0) & (d < NH)
        didx[j * 16:(j + 1) * 16] = jnp.where(ok, d, NH)


def _load_bounds(bnd_hbm, bv, w, half):
    pltpu.sync_copy(bnd_hbm.at[pl.ds((w * NPASS + half) * 8, 8)], bv)
    vecb = bv[0, 0:16]
    return vecb[0], vecb[1]


def _sc_deg_body(dst_hbm, bnd_hbm, out_hbm, draw, didx, bv, ones_v, acc):
    c = lax.axis_index("c")
    s = lax.axis_index("s")
    row0 = s * RPS
    w = c * 16 + s
    for half in range(NPASS):
        _fill_buf(ones_v, 128, 128, 0.0)
        _zero_acc_rows(ones_v, acc, row0)
        _fill_buf(ones_v, K, 128, 1.0)
        plsc.subcore_barrier()
        lo, hi = _load_bounds(bnd_hbm, bv, w, half)

        def chunk(i, _):
            ch = lo + i

            @pl.when(ch < hi)
            def _():
                base = ch * K
                pltpu.sync_copy(dst_hbm.at[pl.ds(base, K)], draw)
                _remap_dst(draw, didx, half)
                pltpu.sync_copy(ones_v, acc.at[didx], add=True)
            return 0
        lax.fori_loop(0, MAXCH32, chunk, 0)
        plsc.subcore_barrier()
        pltpu.sync_copy(acc.at[pl.ds(row0, RPS)],
                        out_hbm.at[pl.ds((c * NPASS + half) * ACC_R + row0, RPS)])


_sc_deg = functools.partial(
    pl.kernel,
    out_type=jax.ShapeDtypeStruct((2 * NPASS * ACC_R, 128), jnp.float32),
    mesh=_MESH,
    scratch_types=[
        pltpu.VMEM((K,), jnp.int32),
        pltpu.VMEM((K,), jnp.int32),
        pltpu.VMEM((8, 16), jnp.int32),
        pltpu.VMEM((K, 128), jnp.float32),
        pltpu.VMEM_SHARED((ACC_R, 128), jnp.float32),
    ],
)(_sc_deg_body)


def _make_sc_agg(feat_split):
    def body(src_hbm, dst_hbm, bnd_hbm, tab_hbm, out_hbm,
             sidx, draw, didx, bv, vals, zb, acc):
        c = lax.axis_index("c")
        s = lax.axis_index("s")
        row0 = s * RPS
        _fill_buf(zb, 128, 128, 0.0)
        if feat_split:
            w = s
            soff = c * E_PAD
        else:
            w = c * 16 + s
            soff = 0

        for half in range(NPASS):
            _zero_acc_rows(zb, acc, row0)
            plsc.subcore_barrier()
            lo, hi = _load_bounds(bnd_hbm, bv, w, half)
            maxch = MAXCH16 if feat_split else MAXCH32

            def chunk(i, _):
                ch = lo + i

                @pl.when(ch < hi)
                def _():
                    base = ch * K
                    pltpu.sync_copy(src_hbm.at[pl.ds(soff + base, K)], sidx)
                    pltpu.sync_copy(dst_hbm.at[pl.ds(base, K)], draw)
                    _remap_dst(draw, didx, half)
                    pltpu.sync_copy(tab_hbm.at[sidx], vals)
                    pltpu.sync_copy(vals, acc.at[didx], add=True)
                return 0
            lax.fori_loop(0, maxch, chunk, 0)
            plsc.subcore_barrier()
            pltpu.sync_copy(acc.at[pl.ds(row0, RPS)],
                            out_hbm.at[pl.ds((c * NPASS + half) * ACC_R + row0, RPS)])

    return functools.partial(
        pl.kernel,
        out_type=jax.ShapeDtypeStruct((2 * NPASS * ACC_R, 128), jnp.float32),
        mesh=_MESH,
        scratch_types=[
            pltpu.VMEM((K,), jnp.int32),
            pltpu.VMEM((K,), jnp.int32),
            pltpu.VMEM((K,), jnp.int32),
            pltpu.VMEM((8, 16), jnp.int32),
            pltpu.VMEM((K, 128), jnp.float32),
            pltpu.VMEM((128, 128), jnp.float32),
            pltpu.VMEM_SHARED((ACC_R, 128), jnp.float32),
        ],
    )(body)


_sc_agg_edge = _make_sc_agg(False)
_sc_agg_feat = _make_sc_agg(True)


def _sc_att1_body(src_hbm, dst_hbm, bnd_hbm, as_hbm, ad_hbm, p_hbm, den_hbm,
                  sidx, draw, didx, bv, arow_s, arow_d, pbuf, pbuf16, acc):
    c = lax.axis_index("c")
    s = lax.axis_index("s")
    row0 = s * RPS
    w = c * 16 + s
    for half in range(NPASS):
        _fill_buf(pbuf, K, 128, 0.0)
        _zero_acc_rows(pbuf, acc, row0)
        plsc.subcore_barrier()
        lo, hi = _load_bounds(bnd_hbm, bv, w, half)

        def chunk(i, _):
            ch = lo + i

            @pl.when(ch < hi)
            def _():
                base = ch * K
                pltpu.sync_copy(src_hbm.at[pl.ds(base, K)], sidx)
                pltpu.sync_copy(dst_hbm.at[pl.ds(base, K)], draw)
                _remap_dst(draw, didx, half)
                pltpu.sync_copy(as_hbm.at[sidx], arow_s)
                pltpu.sync_copy(ad_hbm.at[draw], arow_d)

                def ebody(i2, _):
                    e = arow_s[i2, 0:16] + arow_d[i2, 0:16]
                    e = jnp.where(e >= 0.0, e, 0.2 * e)
                    p = jnp.exp(e)
                    pbuf[i2, 0:16] = p
                    pbuf16[i2, 0:16] = p
                    return 0
                lax.fori_loop(0, K, ebody, 0)
                pltpu.sync_copy(pbuf16, p_hbm.at[pl.ds(base, K)])
                pltpu.sync_copy(pbuf, acc.at[didx], add=True)
            return 0
        lax.fori_loop(0, MAXCH32, chunk, 0)
        plsc.subcore_barrier()
        pltpu.sync_copy(acc.at[pl.ds(row0, RPS)],
                        den_hbm.at[pl.ds((c * NPASS + half) * ACC_R + row0, RPS)])


_sc_att1 = functools.partial(
    pl.kernel,
    out_type=[
        jax.ShapeDtypeStruct((E_PAD, 16), jnp.float32),
        jax.ShapeDtypeStruct((2 * NPASS * ACC_R, 128), jnp.float32),
    ],
    mesh=_MESH,
    scratch_types=[
        pltpu.VMEM((K,), jnp.int32),
        pltpu.VMEM((K,), jnp.int32),
        pltpu.VMEM((K,), jnp.int32),
        pltpu.VMEM((8, 16), jnp.int32),
        pltpu.VMEM((K, 128), jnp.float32),
        pltpu.VMEM((K, 128), jnp.float32),
        pltpu.VMEM((K, 128), jnp.float32),
        pltpu.VMEM((K, 16), jnp.float32),
        pltpu.VMEM_SHARED((ACC_R, 128), jnp.float32),
    ],
)(_sc_att1_body)


def _sc_att2_body(src_hbm, dst_hbm, bnd_hbm, htab_hbm, p_hbm, den_hbm, out_hbm,
                  sidx, draw, didx, bv, hbuf, pbuf, drow, vbuf, acc):
    c = lax.axis_index("c")
    s = lax.axis_index("s")
    row0 = s * RPS
    for half in range(NPASS):
        _fill_buf(vbuf, K, 128, 0.0)
        _zero_acc_rows(vbuf, acc, row0)
        plsc.subcore_barrier()
        lo, hi = _load_bounds(bnd_hbm, bv, s, half)

        def chunk(i0, _):
            ch = lo + i0

            @pl.when(ch < hi)
            def _():
                base = ch * K
                sbase = c * E_PAD + base
                pltpu.sync_copy(src_hbm.at[pl.ds(sbase, K)], sidx)
                pltpu.sync_copy(dst_hbm.at[pl.ds(base, K)], draw)
                _remap_dst(draw, didx, half)
                pltpu.sync_copy(htab_hbm.at[sidx], hbuf)
                pltpu.sync_copy(p_hbm.at[pl.ds(base, K)], pbuf)
                pltpu.sync_copy(den_hbm.at[draw], drow)

                def blend(i, _):
                    cf = (0.25 * pbuf[i, 0:16]) / (drow[i, 0:16] + 1e-16)
                    c0 = cf[0]
                    c1 = cf[1]
                    c2 = cf[2]
                    c3 = cf[3]
                    for j in range(8):
                        v = (c0 * hbuf[i, 0 + j * 16:0 + j * 16 + 16]
                             + c1 * hbuf[i, 128 + j * 16:128 + j * 16 + 16]
                             + c2 * hbuf[i, 256 + j * 16:256 + j * 16 + 16]
                             + c3 * hbuf[i, 384 + j * 16:384 + j * 16 + 16])
                        vbuf[i, j * 16:j * 16 + 16] = v
                    return 0
                lax.fori_loop(0, K, blend, 0)
                pltpu.sync_copy(vbuf, acc.at[didx], add=True)
            return 0
        lax.fori_loop(0, MAXCH16, chunk, 0)
        plsc.subcore_barrier()
        pltpu.sync_copy(acc.at[pl.ds(row0, RPS)],
                        out_hbm.at[pl.ds((c * NPASS + half) * ACC_R + row0, RPS)])


_sc_att2 = functools.partial(
    pl.kernel,
    out_type=jax.ShapeDtypeStruct((2 * NPASS * ACC_R, 128), jnp.float32),
    mesh=_MESH,
    scratch_types=[
        pltpu.VMEM((K,), jnp.int32),
        pltpu.VMEM((K,), jnp.int32),
        pltpu.VMEM((K,), jnp.int32),
        pltpu.VMEM((8, 16), jnp.int32),
        pltpu.VMEM((K, 512), jnp.float32),
        pltpu.VMEM((K, 16), jnp.float32),
        pltpu.VMEM((K, 128), jnp.float32),
        pltpu.VMEM((K, 128), jnp.float32),
        pltpu.VMEM_SHARED((ACC_R, 128), jnp.float32),
    ],
)(_sc_att2_body)


def _bn_relu(h, g, be):
    m = jnp.mean(h, axis=0, keepdims=True)
    d = h - m
    v = jnp.mean(d * d, axis=0, keepdims=True)
    return jnp.maximum(d / jnp.sqrt(v + 1e-5) * g + be, 0.0)


def _tc_first_body(deg_ref, x_ref, w_ref, dinv_ref, out_ref):
    deg = deg_ref[:, :]
    dinv = 1.0 / jnp.sqrt(deg)
    dinv_ref[:, :] = dinv
    hp = dinv * jnp.dot(x_ref[:, :], w_ref[:, :], preferred_element_type=jnp.float32)
    out_ref[0:N, :] = hp
    out_ref[N:N_PAD, :] = jnp.zeros((N_PAD - N, HID), jnp.float32)


def _tc_first(deg, x, W1):
    return pl.pallas_call(
        _tc_first_body,
        out_shape=[
            jax.ShapeDtypeStruct((N, 1), jnp.float32),
            jax.ShapeDtypeStruct((N_PAD, HID), jnp.float32),
        ],
    )(deg, x, W1)


def _make_tc_mid(d_in, d_out):
    def body(agg_ref, dinv_ref, b_ref, g_ref, be_ref, w_ref, out_ref):
        dinv = dinv_ref[:, :]
        h = dinv * agg_ref[:, :] + b_ref[:, :]
        y = _bn_relu(h, g_ref[:, :], be_ref[:, :])
        hp = dinv * jnp.dot(y, w_ref[:, :], preferred_element_type=jnp.float32)
        half = d_out // 2
        out_ref[0, 0:N, :] = hp[:, :half]
        out_ref[1, 0:N, :] = hp[:, half:]
        out_ref[0, N:N_PAD, :] = jnp.zeros((N_PAD - N, half), jnp.float32)
        out_ref[1, N:N_PAD, :] = jnp.zeros((N_PAD - N, half), jnp.float32)

    def run(agg, dinv, b, g, be, Wn):
        return pl.pallas_call(
            body,
            out_shape=jax.ShapeDtypeStruct((2, N_PAD, d_out // 2), jnp.float32),
        )(agg, dinv, b.reshape(1, d_in), g.reshape(1, d_in), be.reshape(1, d_in), Wn)
    return run


_tc_mid2 = _make_tc_mid(HID, OUT)
_tc_mid3 = _make_tc_mid(OUT, OUT)


def _tc_gatin_body(agg_ref, dinv_ref, b_ref, g_ref, be_ref, aw_ref,
                   y_ref, aa_ref):
    dinv = dinv_ref[:, :]
    h = dinv * agg_ref[:, :] + b_ref[:, :]
    y = _bn_relu(h, g_ref[:, :], be_ref[:, :])
    y_ref[:, :] = y
    aa_ref[:, :] = jnp.dot(y, aw_ref[:, :], preferred_element_type=jnp.float32)


def _tc_gatin(agg, dinv, b3, g3, be3, AsAd):
    return pl.pallas_call(
        _tc_gatin_body,
        out_shape=[
            jax.ShapeDtypeStruct((N, OUT), jnp.float32),
            jax.ShapeDtypeStruct((N, 32), jnp.float32),
        ],
    )(agg, dinv, b3.reshape(1, OUT), g3.reshape(1, OUT), be3.reshape(1, OUT), AsAd)


def _tc_hproj_body(y_ref, wg_ref, out_ref):
    out_ref[0, 0:N, :] = jnp.dot(y_ref[:, :], wg_ref[0], preferred_element_type=jnp.float32)
    out_ref[0, N:N_PAD, :] = jnp.zeros((N_PAD - N, 512), jnp.float32)


def _tc_hproj(y3, WgR):
    return pl.pallas_call(
        _tc_hproj_body,
        grid=(2,),
        in_specs=[
            pl.BlockSpec((N, OUT), lambda i: (0, 0)),
            pl.BlockSpec((1, OUT, 512), lambda i: (i, 0, 0)),
        ],
        out_specs=pl.BlockSpec((1, N_PAD, 512), lambda i: (i, 0, 0)),
        out_shape=jax.ShapeDtypeStruct((2, N_PAD, 512), jnp.float32),
    )(y3, WgR)


def _tc_final_body(t_ref, bg_ref, g_ref, be_ref, batch_ref, wfc_ref, bfc_ref,
                   out_ref):
    h = t_ref[:, :] + bg_ref[:, :]
    z = _bn_relu(h, g_ref[:, :], be_ref[:, :])
    b = batch_ref[:, :]
    gids = jax.lax.broadcasted_iota(jnp.int32, (N_GRAPHS, N), 0)
    mask = (b == gids).astype(jnp.float32)
    sums = jnp.dot(mask, z, preferred_element_type=jnp.float32)
    cnt = jnp.sum(mask, axis=1, keepdims=True)
    pooled = sums / jnp.maximum(cnt, 1.0)
    out = jnp.dot(pooled, wfc_ref[:, :], preferred_element_type=jnp.float32)
    out_ref[:, :] = jnp.maximum(out + bfc_ref[:, :], 0.0)


def _tc_final(t, bg, g3, be3, batch, Wfc, bfc):
    return pl.pallas_call(
        _tc_final_body,
        out_shape=jax.ShapeDtypeStruct((N_GRAPHS, FUSED), jnp.float32),
    )(t, bg.reshape(1, GAT_OUT), g3.reshape(1, GAT_OUT), be3.reshape(1, GAT_OUT),
      batch.reshape(1, N), Wfc, bfc.reshape(1, FUSED))


def _core_rows(o, c):
    parts = []
    for q in range(NPASS):
        nrows = min(NH, N - q * NH)
        base = (c * NPASS + q) * ACC_R
        parts.append(o[base:base + nrows])
    return jnp.concatenate(parts, axis=0)


def _recon_edge_split(o):
    return _core_rows(o, 0) + _core_rows(o, 1)


def _recon_feat_split(o):
    return jnp.concatenate([_core_rows(o, 0), _core_rows(o, 1)], axis=1)


def kernel(x, edge_index, batch, W1, b1, g1, be1, W2, b2, g2, be2,
           W3, b3, g3, be3, Wg, att_src, att_dst, bg, Wfc, bfc):
    loop = jnp.arange(N, dtype=edge_index.dtype)
    src = jnp.concatenate([edge_index[0], loop])
    dst = jnp.concatenate([edge_index[1], loop])
    src_p = jnp.full((E_PAD,), N, jnp.int32).at[:E_TOT].set(src)
    dst_p = jnp.full((E_PAD,), TRASH, jnp.int32).at[:E_TOT].set(dst)

    q = dst_p // NH
    oh = (q[:, None] == jnp.arange(NPASS, dtype=q.dtype)).astype(jnp.int32)
    cum = jnp.cumsum(oh, axis=0)
    counts = cum[-1]
    ends = jnp.cumsum(counts)
    starts = ends - counts
    rank = jnp.sum(oh * (starts[None, :] + cum - 1), axis=1)
    src_s = jnp.zeros((E_PAD,), jnp.int32).at[rank].set(src_p, unique_indices=True)
    dst_s = jnp.full((E_PAD,), TRASH, jnp.int32).at[rank].set(dst_p, unique_indices=True)
    src2 = jnp.concatenate([src_s, src_s + N_PAD])
    ch_s = starts // K
    ch_e = (ends + K - 1) // K
    cq = ch_e - ch_s

    def _bounds(nw):
        w = jnp.arange(nw, dtype=jnp.int32)[:, None]
        lo = ch_s[None, :] + (cq[None, :] * w) // nw
        hi = ch_s[None, :] + (cq[None, :] * (w + 1)) // nw
        b = jnp.zeros((nw, NPASS, 8, 16), jnp.int32)
        b = b.at[:, :, 0, 0].set(lo).at[:, :, 0, 1].set(hi)
        return b.reshape(nw * NPASS * 8, 16)

    b16 = _bounds(16)
    b32 = _bounds(32)

    Wg3 = Wg.reshape(OUT, HEADS, GAT_OUT)
    As = jnp.einsum("khd,hd->kh", Wg3, att_src)
    Ad = jnp.einsum("khd,hd->kh", Wg3, att_dst)
    AsAd = jnp.zeros((OUT, 32), jnp.float32).at[:, 0:4].set(As).at[:, 16:20].set(Ad)
    WgR = jnp.stack([Wg3[:, :, :128].reshape(OUT, 512),
                     Wg3[:, :, 128:].reshape(OUT, 512)])

    deg = _recon_edge_split(_sc_deg(dst_s, b32))[:, 0:1]

    dinv, h1p = _tc_first(deg, x, W1)
    agg1 = _recon_edge_split(_sc_agg_edge(src_s, dst_s, b32, h1p))

    h2p = _tc_mid2(agg1, dinv, b1, g1, be1, W2)
    agg2 = _recon_feat_split(_sc_agg_feat(src2, dst_s, b16, h2p.reshape(2 * N_PAD, 128)))

    h3p = _tc_mid3(agg2, dinv, b2, g2, be2, W3)
    agg3 = _recon_feat_split(_sc_agg_feat(src2, dst_s, b16, h3p.reshape(2 * N_PAD, 128)))

    y3, aa = _tc_gatin(agg3, dinv, b3, g3, be3, AsAd)
    as_tab = jnp.zeros((N_PAD, 128), jnp.float32).at[:N, 0:16].set(aa[:, 0:16])
    ad_tab = jnp.zeros((N_PAD, 128), jnp.float32).at[:N, 0:16].set(aa[:, 16:32])

    p_e, den4 = _sc_att1(src_s, dst_s, b32, as_tab, ad_tab)
    den = jnp.zeros((N_PAD, 128), jnp.float32).at[:N].set(_recon_edge_split(den4))

    htab = _tc_hproj(y3, WgR)
    t = _recon_feat_split(
        _sc_att2(src2, dst_s, b16, htab.reshape(2 * N_PAD, 512), p_e, den))

    return _tc_final(t, bg, g3, be3, batch, Wfc, bfc)


# speedup vs baseline: 8.7017x; 1.1753x over previous
import functools

import jax
import jax.numpy as jnp
from jax import lax
from jax.experimental import pallas as pl
from jax.experimental.pallas import tpu as pltpu
from jax.experimental.pallas import tpu_sc as plsc

N = 10000
E = 320000
D_IN = 128
HID = 128
OUT = 2 * HID
GAT_OUT = 256
HEADS = 4
N_GRAPHS = 64
FUSED = 512

N_PAD = 10112
TRASH = 10008
E_TOT = E + N
K = 128
E_PAD = 331776

NPASS = 8
NH = 1264
ACC_R = 1280
RPS = ACC_R // 16
MAXCH32 = (E_PAD // K + 31) // 32 + 1
MAXCH16 = (E_PAD // K + 15) // 16 + 1

_MESH = plsc.VectorSubcoreMesh(core_axis_name="c", subcore_axis_name="s")


def _fill_buf(buf, rows, dh, val):
    zv = jnp.full((16,), val, jnp.float32)
    for j in range(dh // 16):
        def body(i, _, j=j):
            buf[i, j * 16:(j + 1) * 16] = zv
            return 0
        lax.fori_loop(0, rows, body, 0)


def _zero_acc_rows(zbuf, acc, row0):
    pltpu.sync_copy(zbuf.at[pl.ds(0, RPS)], acc.at[pl.ds(row0, RPS)])


def _remap_dst(cidx, didx, half, base=128):
    off = half * NH
    for j in range(8):
        d = cidx[base + j * 16:base + (j + 1) * 16] - off
        ok = (d >= 0) & (d < NH)
        didx[j * 16:(j + 1) * 16] = jnp.where(ok, d, NH)

NCH = E_PAD // K


def _load_bounds(bnd_hbm, bv, w, half):
    pltpu.sync_copy(bnd_hbm.at[pl.ds((w * NPASS + half) * 8, 8)], bv)
    vecb = bv[0, 0:16]
    return vecb[0], vecb[1]


def _sc_deg_body(comb_hbm, bnd_hbm, out_hbm, cidx, didx, bv, ones_v, acc):
    c = lax.axis_index("c")
    s = lax.axis_index("s")
    row0 = s * RPS
    w = c * 16 + s
    for half in range(NPASS):
        _fill_buf(ones_v, 128, 128, 0.0)
        _zero_acc_rows(ones_v, acc, row0)
        _fill_buf(ones_v, K, 128, 1.0)
        plsc.subcore_barrier()
        lo, hi = _load_bounds(bnd_hbm, bv, w, half)

        def chunk(i, _):
            ch = lo + i

            @pl.when(ch < hi)
            def _():
                pltpu.sync_copy(comb_hbm.at[pl.ds(ch * 2 * K + K, K)], cidx)
                _remap_dst(cidx, didx, half, base=0)
                pltpu.sync_copy(ones_v, acc.at[didx], add=True)
            return 0
        lax.fori_loop(0, MAXCH32, chunk, 0)
        plsc.subcore_barrier()
        pltpu.sync_copy(acc.at[pl.ds(row0, RPS)],
                        out_hbm.at[pl.ds((c * NPASS + half) * ACC_R + row0, RPS)])


_sc_deg = functools.partial(
    pl.kernel,
    out_type=jax.ShapeDtypeStruct((2 * NPASS * ACC_R, 128), jnp.float32),
    mesh=_MESH,
    scratch_types=[
        pltpu.VMEM((K,), jnp.int32),
        pltpu.VMEM((K,), jnp.int32),
        pltpu.VMEM((8, 16), jnp.int32),
        pltpu.VMEM((K, 128), jnp.float32),
        pltpu.VMEM_SHARED((ACC_R, 128), jnp.float32),
    ],
)(_sc_deg_body)


def _make_sc_agg(feat_split):
    def body(comb_hbm, bnd_hbm, tab_hbm, out_hbm,
             cidx, didx, bv, vals, zb, acc):
        c = lax.axis_index("c")
        s = lax.axis_index("s")
        row0 = s * RPS
        _fill_buf(zb, 128, 128, 0.0)
        if feat_split:
            w = s
            coff = c * (2 * E_PAD)
        else:
            w = c * 16 + s
            coff = 0

        for half in range(NPASS):
            _zero_acc_rows(zb, acc, row0)
            plsc.subcore_barrier()
            lo, hi = _load_bounds(bnd_hbm, bv, w, half)
            maxch = MAXCH16 if feat_split else MAXCH32

            def chunk(i, _):
                ch = lo + i

                @pl.when(ch < hi)
                def _():
                    pltpu.sync_copy(comb_hbm.at[pl.ds(coff + ch * 2 * K, 2 * K)], cidx)
                    _remap_dst(cidx, didx, half)
                    pltpu.sync_copy(tab_hbm.at[cidx.at[pl.ds(0, K)]], vals)
                    pltpu.sync_copy(vals, acc.at[didx], add=True)
                return 0
            lax.fori_loop(0, maxch, chunk, 0)
            plsc.subcore_barrier()
            pltpu.sync_copy(acc.at[pl.ds(row0, RPS)],
                            out_hbm.at[pl.ds((c * NPASS + half) * ACC_R + row0, RPS)])

    return functools.partial(
        pl.kernel,
        out_type=jax.ShapeDtypeStruct((2 * NPASS * ACC_R, 128), jnp.float32),
        mesh=_MESH,
        scratch_types=[
            pltpu.VMEM((2 * K,), jnp.int32),
            pltpu.VMEM((K,), jnp.int32),
            pltpu.VMEM((8, 16), jnp.int32),
            pltpu.VMEM((K, 128), jnp.float32),
            pltpu.VMEM((128, 128), jnp.float32),
            pltpu.VMEM_SHARED((ACC_R, 128), jnp.float32),
        ],
    )(body)


_sc_agg_edge = _make_sc_agg(False)
_sc_agg_feat = _make_sc_agg(True)


def _sc_att1_body(comb_hbm, bnd_hbm, as_hbm, ad_hbm, p_hbm, den_hbm,
                  cidx, didx, bv, arow_s, arow_d, pbuf, pbuf16, acc):
    c = lax.axis_index("c")
    s = lax.axis_index("s")
    row0 = s * RPS
    w = c * 16 + s
    for half in range(NPASS):
        _fill_buf(pbuf, K, 128, 0.0)
        _zero_acc_rows(pbuf, acc, row0)
        plsc.subcore_barrier()
        lo, hi = _load_bounds(bnd_hbm, bv, w, half)

        def chunk(i, _):
            ch = lo + i

            @pl.when(ch < hi)
            def _():
                base = ch * K
                pltpu.sync_copy(comb_hbm.at[pl.ds(ch * 2 * K, 2 * K)], cidx)
                _remap_dst(cidx, didx, half)
                pltpu.sync_copy(as_hbm.at[cidx.at[pl.ds(0, K)]], arow_s)
                pltpu.sync_copy(ad_hbm.at[cidx.at[pl.ds(K, K)]], arow_d)

                def ebody(i2, _):
                    e = arow_s[i2, 0:16] + arow_d[i2, 0:16]
                    e = jnp.where(e >= 0.0, e, 0.2 * e)
                    p = jnp.exp(e)
                    pbuf[i2, 0:16] = p
                    pbuf16[i2, 0:16] = p
                    return 0
                lax.fori_loop(0, K, ebody, 0)
                pltpu.sync_copy(pbuf16, p_hbm.at[pl.ds(base, K)])
                pltpu.sync_copy(pbuf, acc.at[didx], add=True)
            return 0
        lax.fori_loop(0, MAXCH32, chunk, 0)
        plsc.subcore_barrier()
        pltpu.sync_copy(acc.at[pl.ds(row0, RPS)],
                        den_hbm.at[pl.ds((c * NPASS + half) * ACC_R + row0, RPS)])


_sc_att1 = functools.partial(
    pl.kernel,
    out_type=[
        jax.ShapeDtypeStruct((E_PAD, 16), jnp.float32),
        jax.ShapeDtypeStruct((2 * NPASS * ACC_R, 128), jnp.float32),
    ],
    mesh=_MESH,
    scratch_types=[
        pltpu.VMEM((2 * K,), jnp.int32),
        pltpu.VMEM((K,), jnp.int32),
        pltpu.VMEM((8, 16), jnp.int32),
        pltpu.VMEM((K, 128), jnp.float32),
        pltpu.VMEM((K, 128), jnp.float32),
        pltpu.VMEM((K, 128), jnp.float32),
        pltpu.VMEM((K, 16), jnp.float32),
        pltpu.VMEM_SHARED((ACC_R, 128), jnp.float32),
    ],
)(_sc_att1_body)


def _sc_att2_body(comb_hbm, bnd_hbm, htab_hbm, p_hbm, den_hbm, out_hbm,
                  cidx, didx, bv, hbuf, pbuf, drow, vbuf, acc):
    c = lax.axis_index("c")
    s = lax.axis_index("s")
    row0 = s * RPS
    for half in range(NPASS):
        _fill_buf(vbuf, K, 128, 0.0)
        _zero_acc_rows(vbuf, acc, row0)
        plsc.subcore_barrier()
        lo, hi = _load_bounds(bnd_hbm, bv, s, half)

        def chunk(i0, _):
            ch = lo + i0

            @pl.when(ch < hi)
            def _():
                base = ch * K
                pltpu.sync_copy(comb_hbm.at[pl.ds(c * 2 * E_PAD + ch * 2 * K, 2 * K)], cidx)
                _remap_dst(cidx, didx, half)
                pltpu.sync_copy(htab_hbm.at[cidx.at[pl.ds(0, K)]], hbuf)
                pltpu.sync_copy(p_hbm.at[pl.ds(base, K)], pbuf)
                pltpu.sync_copy(den_hbm.at[cidx.at[pl.ds(K, K)]], drow)

                def blend(i, _):
                    cf = (0.25 * pbuf[i, 0:16]) / (drow[i, 0:16] + 1e-16)
                    c0 = cf[0]
                    c1 = cf[1]
                    c2 = cf[2]
                    c3 = cf[3]
                    for j in range(8):
                        v = (c0 * hbuf[i, 0 + j * 16:0 + j * 16 + 16]
                             + c1 * hbuf[i, 128 + j * 16:128 + j * 16 + 16]
                             + c2 * hbuf[i, 256 + j * 16:256 + j * 16 + 16]
                             + c3 * hbuf[i, 384 + j * 16:384 + j * 16 + 16])
                        vbuf[i, j * 16:j * 16 + 16] = v
                    return 0
                lax.fori_loop(0, K, blend, 0)
                pltpu.sync_copy(vbuf, acc.at[didx], add=True)
            return 0
        lax.fori_loop(0, MAXCH16, chunk, 0)
        plsc.subcore_barrier()
        pltpu.sync_copy(acc.at[pl.ds(row0, RPS)],
                        out_hbm.at[pl.ds((c * NPASS + half) * ACC_R + row0, RPS)])


_sc_att2 = functools.partial(
    pl.kernel,
    out_type=jax.ShapeDtypeStruct((2 * NPASS * ACC_R, 128), jnp.float32),
    mesh=_MESH,
    scratch_types=[
        pltpu.VMEM((2 * K,), jnp.int32),
        pltpu.VMEM((K,), jnp.int32),
        pltpu.VMEM((8, 16), jnp.int32),
        pltpu.VMEM((K, 512), jnp.float32),
        pltpu.VMEM((K, 16), jnp.float32),
        pltpu.VMEM((K, 128), jnp.float32),
        pltpu.VMEM((K, 128), jnp.float32),
        pltpu.VMEM_SHARED((ACC_R, 128), jnp.float32),
    ],
)(_sc_att2_body)


def _bn_relu(h, g, be):
    m = jnp.mean(h, axis=0, keepdims=True)
    d = h - m
    v = jnp.mean(d * d, axis=0, keepdims=True)
    return jnp.maximum(d / jnp.sqrt(v + 1e-5) * g + be, 0.0)


def _tc_first_body(deg_ref, x_ref, w_ref, dinv_ref, out_ref):
    deg = deg_ref[:, :]
    dinv = 1.0 / jnp.sqrt(deg)
    dinv_ref[:, :] = dinv
    hp = dinv * jnp.dot(x_ref[:, :], w_ref[:, :], preferred_element_type=jnp.float32)
    out_ref[0:N, :] = hp
    out_ref[N:N_PAD, :] = jnp.zeros((N_PAD - N, HID), jnp.float32)


def _tc_first(deg, x, W1):
    return pl.pallas_call(
        _tc_first_body,
        out_shape=[
            jax.ShapeDtypeStruct((N, 1), jnp.float32),
            jax.ShapeDtypeStruct((N_PAD, HID), jnp.float32),
        ],
    )(deg, x, W1)


def _make_tc_mid(d_in, d_out):
    def body(agg_ref, dinv_ref, b_ref, g_ref, be_ref, w_ref, out_ref):
        dinv = dinv_ref[:, :]
        h = dinv * agg_ref[:, :] + b_ref[:, :]
        y = _bn_relu(h, g_ref[:, :], be_ref[:, :])
        hp = dinv * jnp.dot(y, w_ref[:, :], preferred_element_type=jnp.float32)
        half = d_out // 2
        out_ref[0, 0:N, :] = hp[:, :half]
        out_ref[1, 0:N, :] = hp[:, half:]
        out_ref[0, N:N_PAD, :] = jnp.zeros((N_PAD - N, half), jnp.float32)
        out_ref[1, N:N_PAD, :] = jnp.zeros((N_PAD - N, half), jnp.float32)

    def run(agg, dinv, b, g, be, Wn):
        return pl.pallas_call(
            body,
            out_shape=jax.ShapeDtypeStruct((2, N_PAD, d_out // 2), jnp.float32),
        )(agg, dinv, b.reshape(1, d_in), g.reshape(1, d_in), be.reshape(1, d_in), Wn)
    return run


_tc_mid2 = _make_tc_mid(HID, OUT)
_tc_mid3 = _make_tc_mid(OUT, OUT)


def _tc_gatin_body(agg_ref, dinv_ref, b_ref, g_ref, be_ref, aw_ref,
                   y_ref, aa_ref):
    dinv = dinv_ref[:, :]
    h = dinv * agg_ref[:, :] + b_ref[:, :]
    y = _bn_relu(h, g_ref[:, :], be_ref[:, :])
    y_ref[:, :] = y
    aa_ref[:, :] = jnp.dot(y, aw_ref[:, :], preferred_element_type=jnp.float32)


def _tc_gatin(agg, dinv, b3, g3, be3, AsAd):
    return pl.pallas_call(
        _tc_gatin_body,
        out_shape=[
            jax.ShapeDtypeStruct((N, OUT), jnp.float32),
            jax.ShapeDtypeStruct((N, 32), jnp.float32),
        ],
    )(agg, dinv, b3.reshape(1, OUT), g3.reshape(1, OUT), be3.reshape(1, OUT), AsAd)


def _tc_hproj_body(y_ref, wg_ref, out_ref):
    out_ref[0, 0:N, :] = jnp.dot(y_ref[:, :], wg_ref[0], preferred_element_type=jnp.float32)
    out_ref[0, N:N_PAD, :] = jnp.zeros((N_PAD - N, 512), jnp.float32)


def _tc_hproj(y3, WgR):
    return pl.pallas_call(
        _tc_hproj_body,
        grid=(2,),
        in_specs=[
            pl.BlockSpec((N, OUT), lambda i: (0, 0)),
            pl.BlockSpec((1, OUT, 512), lambda i: (i, 0, 0)),
        ],
        out_specs=pl.BlockSpec((1, N_PAD, 512), lambda i: (i, 0, 0)),
        out_shape=jax.ShapeDtypeStruct((2, N_PAD, 512), jnp.float32),
    )(y3, WgR)


def _tc_final_body(t_ref, bg_ref, g_ref, be_ref, batch_ref, wfc_ref, bfc_ref,
                   out_ref):
    h = t_ref[:, :] + bg_ref[:, :]
    z = _bn_relu(h, g_ref[:, :], be_ref[:, :])
    b = batch_ref[:, :]
    gids = jax.lax.broadcasted_iota(jnp.int32, (N_GRAPHS, N), 0)
    mask = (b == gids).astype(jnp.float32)
    sums = jnp.dot(mask, z, preferred_element_type=jnp.float32)
    cnt = jnp.sum(mask, axis=1, keepdims=True)
    pooled = sums / jnp.maximum(cnt, 1.0)
    out = jnp.dot(pooled, wfc_ref[:, :], preferred_element_type=jnp.float32)
    out_ref[:, :] = jnp.maximum(out + bfc_ref[:, :], 0.0)


def _tc_final(t, bg, g3, be3, batch, Wfc, bfc):
    return pl.pallas_call(
        _tc_final_body,
        out_shape=jax.ShapeDtypeStruct((N_GRAPHS, FUSED), jnp.float32),
    )(t, bg.reshape(1, GAT_OUT), g3.reshape(1, GAT_OUT), be3.reshape(1, GAT_OUT),
      batch.reshape(1, N), Wfc, bfc.reshape(1, FUSED))


def _core_rows(o, c):
    parts = []
    for q in range(NPASS):
        nrows = min(NH, N - q * NH)
        base = (c * NPASS + q) * ACC_R
        parts.append(o[base:base + nrows])
    return jnp.concatenate(parts, axis=0)


def _recon_edge_split(o):
    return _core_rows(o, 0) + _core_rows(o, 1)


def _recon_feat_split(o):
    return jnp.concatenate([_core_rows(o, 0), _core_rows(o, 1)], axis=1)


def kernel(x, edge_index, batch, W1, b1, g1, be1, W2, b2, g2, be2,
           W3, b3, g3, be3, Wg, att_src, att_dst, bg, Wfc, bfc):
    loop = jnp.arange(N, dtype=edge_index.dtype)
    src = jnp.concatenate([edge_index[0], loop])
    dst = jnp.concatenate([edge_index[1], loop])
    src_p = jnp.full((E_PAD,), N, jnp.int32).at[:E_TOT].set(src)
    dst_p = jnp.full((E_PAD,), TRASH, jnp.int32).at[:E_TOT].set(dst)

    q = dst_p // NH
    oh = (q[:, None] == jnp.arange(NPASS, dtype=q.dtype)).astype(jnp.int32)
    cum = jnp.cumsum(oh, axis=0)
    counts = cum[-1]
    ends = jnp.cumsum(counts)
    starts = ends - counts
    rank = jnp.sum(oh * (starts[None, :] + cum - 1), axis=1)
    packed = src_p * 16384 + dst_p
    packed_s = jnp.full((E_PAD,), N * 16384 + TRASH, jnp.int32
                        ).at[rank].set(packed, unique_indices=True)
    src_s = packed_s // 16384
    dst_s = packed_s % 16384
    src_r = src_s.reshape(E_PAD // K, K)
    dst_r = dst_s.reshape(1, E_PAD // K, K)
    comb = jnp.concatenate(
        [jnp.stack([src_r, src_r + N_PAD]), jnp.broadcast_to(dst_r, (2, E_PAD // K, K))],
        axis=2).reshape(2 * E_PAD // K * 2 * K)
    ch_s = starts // K
    ch_e = (ends + K - 1) // K
    cq = ch_e - ch_s

    def _bounds(nw):
        w = jnp.arange(nw, dtype=jnp.int32)[:, None]
        lo = ch_s[None, :] + (cq[None, :] * w) // nw
        hi = ch_s[None, :] + (cq[None, :] * (w + 1)) // nw
        b = jnp.zeros((nw, NPASS, 8, 16), jnp.int32)
        b = b.at[:, :, 0, 0].set(lo).at[:, :, 0, 1].set(hi)
        return b.reshape(nw * NPASS * 8, 16)

    b16 = _bounds(16)
    b32 = _bounds(32)

    Wg3 = Wg.reshape(OUT, HEADS, GAT_OUT)
    As = jnp.einsum("khd,hd->kh", Wg3, att_src)
    Ad = jnp.einsum("khd,hd->kh", Wg3, att_dst)
    AsAd = jnp.zeros((OUT, 32), jnp.float32).at[:, 0:4].set(As).at[:, 16:20].set(Ad)
    WgR = jnp.stack([Wg3[:, :, :128].reshape(OUT, 512),
                     Wg3[:, :, 128:].reshape(OUT, 512)])

    deg = _recon_edge_split(_sc_deg(dst_s, b32))[:, 0:1]

    dinv, h1p = _tc_first(deg, x, W1)
    agg1 = _recon_edge_split(_sc_agg_edge(comb, b32, h1p))

    h2p = _tc_mid2(agg1, dinv, b1, g1, be1, W2)
    agg2 = _recon_feat_split(_sc_agg_feat(comb, b16, h2p.reshape(2 * N_PAD, 128)))

    h3p = _tc_mid3(agg2, dinv, b2, g2, be2, W3)
    agg3 = _recon_feat_split(_sc_agg_feat(comb, b16, h3p.reshape(2 * N_PAD, 128)))

    y3, aa = _tc_gatin(agg3, dinv, b3, g3, be3, AsAd)
    as_tab = jnp.zeros((N_PAD, 128), jnp.float32).at[:N, 0:16].set(aa[:, 0:16])
    ad_tab = jnp.zeros((N_PAD, 128), jnp.float32).at[:N, 0:16].set(aa[:, 16:32])

    p_e, den4 = _sc_att1(comb, b32, as_tab, ad_tab)
    den = jnp.zeros((N_PAD, 128), jnp.float32).at[:N].set(_recon_edge_split(den4))

    htab = _tc_hproj(y3, WgR)
    t = _recon_feat_split(
        _sc_att2(comb, b16, htab.reshape(2 * N_PAD, 512), p_e, den))

    return _tc_final(t, bg, g3, be3, batch, Wfc, bfc)


# speedup vs baseline: 9.0306x; 1.0378x over previous
import functools

import jax
import jax.numpy as jnp
from jax import lax
from jax.experimental import pallas as pl
from jax.experimental.pallas import tpu as pltpu
from jax.experimental.pallas import tpu_sc as plsc

N = 10000
E = 320000
D_IN = 128
HID = 128
OUT = 2 * HID
GAT_OUT = 256
HEADS = 4
N_GRAPHS = 64
FUSED = 512

N_PAD = 10112
TRASH = 10008
E_TOT = E + N
K = 128
E_PAD = 331776

NPASS = 8
NH = 1264
ACC_R = 1280
RPS = ACC_R // 16
MAXCH32 = (E_PAD // K + 31) // 32 + 1
MAXCH16 = (E_PAD // K + 15) // 16 + 1

_MESH = plsc.VectorSubcoreMesh(core_axis_name="c", subcore_axis_name="s")


def _fill_buf(buf, rows, dh, val):
    zv = jnp.full((16,), val, jnp.float32)
    for j in range(dh // 16):
        def body(i, _, j=j):
            buf[i, j * 16:(j + 1) * 16] = zv
            return 0
        lax.fori_loop(0, rows, body, 0)


def _zero_acc_rows(zbuf, acc, row0):
    pltpu.sync_copy(zbuf.at[pl.ds(0, RPS)], acc.at[pl.ds(row0, RPS)])


def _remap_dst(cidx, didx, half, base=128):
    off = half * NH
    for j in range(8):
        d = cidx[base + j * 16:base + (j + 1) * 16] - off
        ok = (d >= 0) & (d < NH)
        didx[j * 16:(j + 1) * 16] = jnp.where(ok, d, NH)

NCH = E_PAD // K


def _load_bounds(bnd_hbm, bv, w, half):
    pltpu.sync_copy(bnd_hbm.at[pl.ds((w * NPASS + half) * 8, 8)], bv)
    vecb = bv[0, 0:16]
    return vecb[0], vecb[1]


def _sc_deg_body(comb_hbm, bnd_hbm, out_hbm, cidx, didx, bv, ones_v, acc):
    c = lax.axis_index("c")
    s = lax.axis_index("s")
    row0 = s * RPS
    w = c * 16 + s
    for half in range(NPASS):
        _fill_buf(ones_v, 128, 128, 0.0)
        _zero_acc_rows(ones_v, acc, row0)
        _fill_buf(ones_v, K, 128, 1.0)
        plsc.subcore_barrier()
        lo, hi = _load_bounds(bnd_hbm, bv, w, half)

        def chunk(i, _):
            ch = lo + i

            @pl.when(ch < hi)
            def _():
                pltpu.sync_copy(comb_hbm.at[pl.ds(ch * 2 * K + K, K)], cidx)
                _remap_dst(cidx, didx, half, base=0)
                pltpu.sync_copy(ones_v, acc.at[didx], add=True)
            return 0
        lax.fori_loop(0, MAXCH32, chunk, 0)
        plsc.subcore_barrier()
        pltpu.sync_copy(acc.at[pl.ds(row0, RPS)],
                        out_hbm.at[pl.ds((c * NPASS + half) * ACC_R + row0, RPS)])


_sc_deg = functools.partial(
    pl.kernel,
    out_type=jax.ShapeDtypeStruct((2 * NPASS * ACC_R, 128), jnp.float32),
    mesh=_MESH,
    scratch_types=[
        pltpu.VMEM((K,), jnp.int32),
        pltpu.VMEM((K,), jnp.int32),
        pltpu.VMEM((8, 16), jnp.int32),
        pltpu.VMEM((K, 128), jnp.float32),
        pltpu.VMEM_SHARED((ACC_R, 128), jnp.float32),
    ],
)(_sc_deg_body)


def _make_sc_agg(feat_split):
    def body(comb_hbm, bnd_hbm, tab_hbm, out_hbm,
             cidx0, cidx1, didx, bv, vals0, vals1, zb, acc, sem0, sem1):
        c = lax.axis_index("c")
        s = lax.axis_index("s")
        row0 = s * RPS
        _fill_buf(zb, 128, 128, 0.0)
        if feat_split:
            w = s
            coff = c * (2 * E_PAD)
        else:
            w = c * 16 + s
            coff = 0

        bufs = ((cidx0, vals0, sem0), (cidx1, vals1, sem1))
        for half in range(NPASS):
            _zero_acc_rows(zb, acc, row0)
            plsc.subcore_barrier()
            lo, hi = _load_bounds(bnd_hbm, bv, w, half)
            maxch = MAXCH16 if feat_split else MAXCH32

            @pl.when(lo < hi)
            def _():
                pltpu.sync_copy(comb_hbm.at[pl.ds(coff + lo * 2 * K, 2 * K)], cidx0)
                pltpu.async_copy(tab_hbm.at[cidx0.at[pl.ds(0, K)]], vals0, sem0)

            def pair(ip, _):
                for b in range(2):
                    cidx_c, vals_c, sem_c = bufs[b]
                    cidx_n, vals_n, sem_n = bufs[1 - b]
                    ch = lo + ip * 2 + b

                    @pl.when(ch < hi)
                    def _(b=b, ch=ch, cidx_c=cidx_c, vals_c=vals_c, sem_c=sem_c,
                          cidx_n=cidx_n, vals_n=vals_n, sem_n=sem_n):
                        pltpu.make_async_copy(
                            tab_hbm.at[cidx_c.at[pl.ds(0, K)]], vals_c, sem_c).wait()

                        @pl.when(ch + 1 < hi)
                        def _():
                            pltpu.sync_copy(
                                comb_hbm.at[pl.ds(coff + (ch + 1) * 2 * K, 2 * K)],
                                cidx_n)
                            pltpu.async_copy(
                                tab_hbm.at[cidx_n.at[pl.ds(0, K)]], vals_n, sem_n)
                        _remap_dst(cidx_c, didx, half)
                        pltpu.sync_copy(vals_c, acc.at[didx], add=True)
                return 0
            lax.fori_loop(0, (maxch + 1) // 2, pair, 0)
            plsc.subcore_barrier()
            pltpu.sync_copy(acc.at[pl.ds(row0, RPS)],
                            out_hbm.at[pl.ds((c * NPASS + half) * ACC_R + row0, RPS)])

    return functools.partial(
        pl.kernel,
        out_type=jax.ShapeDtypeStruct((2 * NPASS * ACC_R, 128), jnp.float32),
        mesh=_MESH,
        scratch_types=[
            pltpu.VMEM((2 * K,), jnp.int32),
            pltpu.VMEM((2 * K,), jnp.int32),
            pltpu.VMEM((K,), jnp.int32),
            pltpu.VMEM((8, 16), jnp.int32),
            pltpu.VMEM((K, 128), jnp.float32),
            pltpu.VMEM((K, 128), jnp.float32),
            pltpu.VMEM((128, 128), jnp.float32),
            pltpu.VMEM_SHARED((ACC_R, 128), jnp.float32),
            pltpu.SemaphoreType.DMA,
            pltpu.SemaphoreType.DMA,
        ],
    )(body)


_sc_agg_edge = _make_sc_agg(False)
_sc_agg_feat = _make_sc_agg(True)


def _sc_att1_body(comb_hbm, bnd_hbm, as_hbm, ad_hbm, p_hbm, den_hbm,
                  cidx, didx, bv, arow_s, arow_d, pbuf, pbuf16, acc):
    c = lax.axis_index("c")
    s = lax.axis_index("s")
    row0 = s * RPS
    w = c * 16 + s
    for half in range(NPASS):
        _fill_buf(pbuf, K, 128, 0.0)
        _zero_acc_rows(pbuf, acc, row0)
        plsc.subcore_barrier()
        lo, hi = _load_bounds(bnd_hbm, bv, w, half)

        def chunk(i, _):
            ch = lo + i

            @pl.when(ch < hi)
            def _():
                base = ch * K
                pltpu.sync_copy(comb_hbm.at[pl.ds(ch * 2 * K, 2 * K)], cidx)
                _remap_dst(cidx, didx, half)
                pltpu.sync_copy(as_hbm.at[cidx.at[pl.ds(0, K)]], arow_s)
                pltpu.sync_copy(ad_hbm.at[cidx.at[pl.ds(K, K)]], arow_d)

                def ebody(i2, _):
                    e = arow_s[i2, 0:16] + arow_d[i2, 0:16]
                    e = jnp.where(e >= 0.0, e, 0.2 * e)
                    p = jnp.exp(e)
                    pbuf[i2, 0:16] = p
                    pbuf16[i2, 0:16] = p
                    return 0
                lax.fori_loop(0, K, ebody, 0)
                pltpu.sync_copy(pbuf16, p_hbm.at[pl.ds(base, K)])
                pltpu.sync_copy(pbuf, acc.at[didx], add=True)
            return 0
        lax.fori_loop(0, MAXCH32, chunk, 0)
        plsc.subcore_barrier()
        pltpu.sync_copy(acc.at[pl.ds(row0, RPS)],
                        den_hbm.at[pl.ds((c * NPASS + half) * ACC_R + row0, RPS)])


_sc_att1 = functools.partial(
    pl.kernel,
    out_type=[
        jax.ShapeDtypeStruct((E_PAD, 16), jnp.float32),
        jax.ShapeDtypeStruct((2 * NPASS * ACC_R, 128), jnp.float32),
    ],
    mesh=_MESH,
    scratch_types=[
        pltpu.VMEM((2 * K,), jnp.int32),
        pltpu.VMEM((K,), jnp.int32),
        pltpu.VMEM((8, 16), jnp.int32),
        pltpu.VMEM((K, 128), jnp.float32),
        pltpu.VMEM((K, 128), jnp.float32),
        pltpu.VMEM((K, 128), jnp.float32),
        pltpu.VMEM((K, 16), jnp.float32),
        pltpu.VMEM_SHARED((ACC_R, 128), jnp.float32),
    ],
)(_sc_att1_body)


def _sc_att2_body(comb_hbm, bnd_hbm, htab_hbm, p_hbm, den_hbm, out_hbm,
                  cidx, didx, bv, hbuf, pbuf, drow, vbuf, acc):
    c = lax.axis_index("c")
    s = lax.axis_index("s")
    row0 = s * RPS
    for half in range(NPASS):
        _fill_buf(vbuf, K, 128, 0.0)
        _zero_acc_rows(vbuf, acc, row0)
        plsc.subcore_barrier()
        lo, hi = _load_bounds(bnd_hbm, bv, s, half)

        def chunk(i0, _):
            ch = lo + i0

            @pl.when(ch < hi)
            def _():
                base = ch * K
                pltpu.sync_copy(comb_hbm.at[pl.ds(c * 2 * E_PAD + ch * 2 * K, 2 * K)], cidx)
                _remap_dst(cidx, didx, half)
                pltpu.sync_copy(htab_hbm.at[cidx.at[pl.ds(0, K)]], hbuf)
                pltpu.sync_copy(p_hbm.at[pl.ds(base, K)], pbuf)
                pltpu.sync_copy(den_hbm.at[cidx.at[pl.ds(K, K)]], drow)

                def blend(i, _):
                    cf = (0.25 * pbuf[i, 0:16]) / (drow[i, 0:16] + 1e-16)
                    c0 = cf[0]
                    c1 = cf[1]
                    c2 = cf[2]
                    c3 = cf[3]
                    for j in range(8):
                        v = (c0 * hbuf[i, 0 + j * 16:0 + j * 16 + 16]
                             + c1 * hbuf[i, 128 + j * 16:128 + j * 16 + 16]
                             + c2 * hbuf[i, 256 + j * 16:256 + j * 16 + 16]
                             + c3 * hbuf[i, 384 + j * 16:384 + j * 16 + 16])
                        vbuf[i, j * 16:j * 16 + 16] = v
                    return 0
                lax.fori_loop(0, K, blend, 0)
                pltpu.sync_copy(vbuf, acc.at[didx], add=True)
            return 0
        lax.fori_loop(0, MAXCH16, chunk, 0)
        plsc.subcore_barrier()
        pltpu.sync_copy(acc.at[pl.ds(row0, RPS)],
                        out_hbm.at[pl.ds((c * NPASS + half) * ACC_R + row0, RPS)])


_sc_att2 = functools.partial(
    pl.kernel,
    out_type=jax.ShapeDtypeStruct((2 * NPASS * ACC_R, 128), jnp.float32),
    mesh=_MESH,
    scratch_types=[
        pltpu.VMEM((2 * K,), jnp.int32),
        pltpu.VMEM((K,), jnp.int32),
        pltpu.VMEM((8, 16), jnp.int32),
        pltpu.VMEM((K, 512), jnp.float32),
        pltpu.VMEM((K, 16), jnp.float32),
        pltpu.VMEM((K, 128), jnp.float32),
        pltpu.VMEM((K, 128), jnp.float32),
        pltpu.VMEM_SHARED((ACC_R, 128), jnp.float32),
    ],
)(_sc_att2_body)


def _bn_relu(h, g, be):
    m = jnp.mean(h, axis=0, keepdims=True)
    d = h - m
    v = jnp.mean(d * d, axis=0, keepdims=True)
    return jnp.maximum(d / jnp.sqrt(v + 1e-5) * g + be, 0.0)


def _tc_first_body(deg_ref, x_ref, w_ref, dinv_ref, out_ref):
    deg = deg_ref[:, :]
    dinv = 1.0 / jnp.sqrt(deg)
    dinv_ref[:, :] = dinv
    hp = dinv * jnp.dot(x_ref[:, :], w_ref[:, :], preferred_element_type=jnp.float32)
    out_ref[0:N, :] = hp
    out_ref[N:N_PAD, :] = jnp.zeros((N_PAD - N, HID), jnp.float32)


def _tc_first(deg, x, W1):
    return pl.pallas_call(
        _tc_first_body,
        out_shape=[
            jax.ShapeDtypeStruct((N, 1), jnp.float32),
            jax.ShapeDtypeStruct((N_PAD, HID), jnp.float32),
        ],
    )(deg, x, W1)


def _make_tc_mid(d_in, d_out):
    def body(agg_ref, dinv_ref, b_ref, g_ref, be_ref, w_ref, out_ref):
        dinv = dinv_ref[:, :]
        h = dinv * agg_ref[:, :] + b_ref[:, :]
        y = _bn_relu(h, g_ref[:, :], be_ref[:, :])
        hp = dinv * jnp.dot(y, w_ref[:, :], preferred_element_type=jnp.float32)
        half = d_out // 2
        out_ref[0, 0:N, :] = hp[:, :half]
        out_ref[1, 0:N, :] = hp[:, half:]
        out_ref[0, N:N_PAD, :] = jnp.zeros((N_PAD - N, half), jnp.float32)
        out_ref[1, N:N_PAD, :] = jnp.zeros((N_PAD - N, half), jnp.float32)

    def run(agg, dinv, b, g, be, Wn):
        return pl.pallas_call(
            body,
            out_shape=jax.ShapeDtypeStruct((2, N_PAD, d_out // 2), jnp.float32),
        )(agg, dinv, b.reshape(1, d_in), g.reshape(1, d_in), be.reshape(1, d_in), Wn)
    return run


_tc_mid2 = _make_tc_mid(HID, OUT)
_tc_mid3 = _make_tc_mid(OUT, OUT)


def _tc_gatin_body(agg_ref, dinv_ref, b_ref, g_ref, be_ref, aw_ref,
                   y_ref, aa_ref):
    dinv = dinv_ref[:, :]
    h = dinv * agg_ref[:, :] + b_ref[:, :]
    y = _bn_relu(h, g_ref[:, :], be_ref[:, :])
    y_ref[:, :] = y
    aa_ref[:, :] = jnp.dot(y, aw_ref[:, :], preferred_element_type=jnp.float32)


def _tc_gatin(agg, dinv, b3, g3, be3, AsAd):
    return pl.pallas_call(
        _tc_gatin_body,
        out_shape=[
            jax.ShapeDtypeStruct((N, OUT), jnp.float32),
            jax.ShapeDtypeStruct((N, 32), jnp.float32),
        ],
    )(agg, dinv, b3.reshape(1, OUT), g3.reshape(1, OUT), be3.reshape(1, OUT), AsAd)


def _tc_hproj_body(y_ref, wg_ref, out_ref):
    out_ref[0, 0:N, :] = jnp.dot(y_ref[:, :], wg_ref[0], preferred_element_type=jnp.float32)
    out_ref[0, N:N_PAD, :] = jnp.zeros((N_PAD - N, 512), jnp.float32)


def _tc_hproj(y3, WgR):
    return pl.pallas_call(
        _tc_hproj_body,
        grid=(2,),
        in_specs=[
            pl.BlockSpec((N, OUT), lambda i: (0, 0)),
            pl.BlockSpec((1, OUT, 512), lambda i: (i, 0, 0)),
        ],
        out_specs=pl.BlockSpec((1, N_PAD, 512), lambda i: (i, 0, 0)),
        out_shape=jax.ShapeDtypeStruct((2, N_PAD, 512), jnp.float32),
    )(y3, WgR)


def _tc_final_body(t_ref, bg_ref, g_ref, be_ref, batch_ref, wfc_ref, bfc_ref,
                   out_ref):
    h = t_ref[:, :] + bg_ref[:, :]
    z = _bn_relu(h, g_ref[:, :], be_ref[:, :])
    b = batch_ref[:, :]
    gids = jax.lax.broadcasted_iota(jnp.int32, (N_GRAPHS, N), 0)
    mask = (b == gids).astype(jnp.float32)
    sums = jnp.dot(mask, z, preferred_element_type=jnp.float32)
    cnt = jnp.sum(mask, axis=1, keepdims=True)
    pooled = sums / jnp.maximum(cnt, 1.0)
    out = jnp.dot(pooled, wfc_ref[:, :], preferred_element_type=jnp.float32)
    out_ref[:, :] = jnp.maximum(out + bfc_ref[:, :], 0.0)


def _tc_final(t, bg, g3, be3, batch, Wfc, bfc):
    return pl.pallas_call(
        _tc_final_body,
        out_shape=jax.ShapeDtypeStruct((N_GRAPHS, FUSED), jnp.float32),
    )(t, bg.reshape(1, GAT_OUT), g3.reshape(1, GAT_OUT), be3.reshape(1, GAT_OUT),
      batch.reshape(1, N), Wfc, bfc.reshape(1, FUSED))


def _core_rows(o, c):
    parts = []
    for q in range(NPASS):
        nrows = min(NH, N - q * NH)
        base = (c * NPASS + q) * ACC_R
        parts.append(o[base:base + nrows])
    return jnp.concatenate(parts, axis=0)


def _recon_edge_split(o):
    return _core_rows(o, 0) + _core_rows(o, 1)


def _recon_feat_split(o):
    return jnp.concatenate([_core_rows(o, 0), _core_rows(o, 1)], axis=1)


def kernel(x, edge_index, batch, W1, b1, g1, be1, W2, b2, g2, be2,
           W3, b3, g3, be3, Wg, att_src, att_dst, bg, Wfc, bfc):
    loop = jnp.arange(N, dtype=edge_index.dtype)
    src = jnp.concatenate([edge_index[0], loop])
    dst = jnp.concatenate([edge_index[1], loop])
    src_p = jnp.full((E_PAD,), N, jnp.int32).at[:E_TOT].set(src)
    dst_p = jnp.full((E_PAD,), TRASH, jnp.int32).at[:E_TOT].set(dst)

    q = dst_p // NH
    oh = (q[:, None] == jnp.arange(NPASS, dtype=q.dtype)).astype(jnp.int32)
    cum = jnp.cumsum(oh, axis=0)
    counts = cum[-1]
    ends = jnp.cumsum(counts)
    starts = ends - counts
    rank = jnp.sum(oh * (starts[None, :] + cum - 1), axis=1)
    packed = src_p * 16384 + dst_p
    packed_s = jnp.full((E_PAD,), N * 16384 + TRASH, jnp.int32
                        ).at[rank].set(packed, unique_indices=True)
    src_s = packed_s // 16384
    dst_s = packed_s % 16384
    src_r = src_s.reshape(E_PAD // K, K)
    dst_r = dst_s.reshape(1, E_PAD // K, K)
    comb = jnp.concatenate(
        [jnp.stack([src_r, src_r + N_PAD]), jnp.broadcast_to(dst_r, (2, E_PAD // K, K))],
        axis=2).reshape(2 * E_PAD // K * 2 * K)
    ch_s = starts // K
    ch_e = (ends + K - 1) // K
    cq = ch_e - ch_s

    def _bounds(nw):
        w = jnp.arange(nw, dtype=jnp.int32)[:, None]
        lo = ch_s[None, :] + (cq[None, :] * w) // nw
        hi = ch_s[None, :] + (cq[None, :] * (w + 1)) // nw
        b = jnp.zeros((nw, NPASS, 8, 16), jnp.int32)
        b = b.at[:, :, 0, 0].set(lo).at[:, :, 0, 1].set(hi)
        return b.reshape(nw * NPASS * 8, 16)

    b16 = _bounds(16)
    b32 = _bounds(32)

    Wg3 = Wg.reshape(OUT, HEADS, GAT_OUT)
    As = jnp.einsum("khd,hd->kh", Wg3, att_src)
    Ad = jnp.einsum("khd,hd->kh", Wg3, att_dst)
    AsAd = jnp.zeros((OUT, 32), jnp.float32).at[:, 0:4].set(As).at[:, 16:20].set(Ad)
    WgR = jnp.stack([Wg3[:, :, :128].reshape(OUT, 512),
                     Wg3[:, :, 128:].reshape(OUT, 512)])

    deg = _recon_edge_split(_sc_deg(dst_s, b32))[:, 0:1]

    dinv, h1p = _tc_first(deg, x, W1)
    agg1 = _recon_edge_split(_sc_agg_edge(comb, b32, h1p))

    h2p = _tc_mid2(agg1, dinv, b1, g1, be1, W2)
    agg2 = _recon_feat_split(_sc_agg_feat(comb, b16, h2p.reshape(2 * N_PAD, 128)))

    h3p = _tc_mid3(agg2, dinv, b2, g2, be2, W3)
    agg3 = _recon_feat_split(_sc_agg_feat(comb, b16, h3p.reshape(2 * N_PAD, 128)))

    y3, aa = _tc_gatin(agg3, dinv, b3, g3, be3, AsAd)
    as_tab = jnp.zeros((N_PAD, 128), jnp.float32).at[:N, 0:16].set(aa[:, 0:16])
    ad_tab = jnp.zeros((N_PAD, 128), jnp.float32).at[:N, 0:16].set(aa[:, 16:32])

    p_e, den4 = _sc_att1(comb, b32, as_tab, ad_tab)
    den = jnp.zeros((N_PAD, 128), jnp.float32).at[:N].set(_recon_edge_split(den4))

    htab = _tc_hproj(y3, WgR)
    t = _recon_feat_split(
        _sc_att2(comb, b16, htab.reshape(2 * N_PAD, 512), p_e, den))

    return _tc_final(t, bg, g3, be3, batch, Wfc, bfc)
